# Initial kernel scaffold; baseline (speedup 1.0000x reference)
import jax
import jax.numpy as jnp
from jax.experimental import pallas as pl


def kernel(x, edge_index, batch, W1, b1, W2, b2, W3, b3, Wl1, bl1, Wl2, bl2):
    raise NotImplementedError("write your pallas kernel here")


# speedup vs baseline: 16.8297x; 16.8297x over previous
import functools

import jax
import jax.numpy as jnp
from jax import lax
from jax.experimental import pallas as pl
from jax.experimental.pallas import tpu as pltpu
from jax.experimental.pallas import tpu_sc as plsc

_NUM_GRAPHS = 64
_NC = 2
_NS = 16
_NW = _NC * _NS
_C = 128


def _mesh():
    return plsc.VectorSubcoreMesh(core_axis_name="c", subcore_axis_name="s")


def _zero_1d(ref, n):
    def body(i, carry):
        ref[pl.ds(i * 16, 16)] = jnp.zeros((16,), jnp.float32)
        return carry
    lax.fori_loop(0, n // 16, body, 0)


def _ones_1d(ref, n):
    def body(i, carry):
        ref[pl.ds(i * 16, 16)] = jnp.ones((16,), jnp.float32)
        return carry
    lax.fori_loop(0, n // 16, body, 0)


def _sc_deg(dst, n_pad):
    e = dst.shape[0]
    assert e % _NW == 0
    epw = e // _NW
    full = epw // _C
    tail = epw - full * _C
    per_tile = n_pad // _NS

    @functools.partial(
        pl.kernel,
        out_type=jax.ShapeDtypeStruct((_NC, n_pad), jnp.float32),
        mesh=_mesh(),
        scratch_types=[
            pltpu.VMEM((_C,), jnp.int32),
            pltpu.VMEM((16,), jnp.int32),
            pltpu.VMEM((_C,), jnp.float32),
            pltpu.VMEM((16,), jnp.float32),
            pltpu.VMEM((per_tile,), jnp.float32),
            pltpu.VMEM_SHARED((n_pad,), jnp.float32),
        ],
    )
    def k(dst_hbm, out_hbm, idx_v, idx_t, ones_v, ones_t, stage_v, acc_sh):
        c = lax.axis_index("c")
        s = lax.axis_index("s")
        wid = s * _NC + c
        _ones_1d(ones_v, _C)
        _ones_1d(ones_t, 16)
        _zero_1d(stage_v, per_tile)
        pltpu.sync_copy(stage_v, acc_sh.at[pl.ds(s * per_tile, per_tile)])
        plsc.subcore_barrier()
        base = wid * epw

        def step(i, carry):
            pltpu.sync_copy(dst_hbm.at[pl.ds(base + i * _C, _C)], idx_v)
            pltpu.sync_copy(ones_v, acc_sh.at[idx_v], add=True)
            return carry

        lax.fori_loop(0, full, step, 0)
        if tail:
            pltpu.sync_copy(dst_hbm.at[pl.ds(base + full * _C, tail)], idx_t)
            pltpu.sync_copy(ones_t, acc_sh.at[idx_t], add=True)
        plsc.subcore_barrier()
        pltpu.sync_copy(acc_sh.at[pl.ds(s * per_tile, per_tile)], stage_v)
        pltpu.sync_copy(stage_v, out_hbm.at[c, pl.ds(s * per_tile, per_tile)])

    return k(dst)


def _sc_propagate(g, src, dst, n_pad):
    n, h = g.shape
    e = src.shape[0]
    assert e % _NW == 0
    epw = e // _NW
    full = epw // _C
    tail = epw - full * _C
    rows_per_tile = n_pad // _NS
    stage_rows = 1
    for cand in range(1, 129):
        if rows_per_tile % cand == 0:
            stage_rows = cand
    n_stage = rows_per_tile // stage_rows

    @functools.partial(
        pl.kernel,
        out_type=jax.ShapeDtypeStruct((_NC, n_pad, h), jnp.float32),
        mesh=_mesh(),
        compiler_params=pltpu.CompilerParams(use_tc_tiling_on_sc=False),
        scratch_types=[
            pltpu.VMEM((_C,), jnp.int32),
            pltpu.VMEM((_C,), jnp.int32),
            pltpu.VMEM((16,), jnp.int32),
            pltpu.VMEM((16,), jnp.int32),
            pltpu.VMEM((_C, h), jnp.float32),
            pltpu.VMEM((16, h), jnp.float32),
            pltpu.VMEM((stage_rows, h), jnp.float32),
            pltpu.SemaphoreType.DMA,
            pltpu.VMEM_SHARED((n_pad, h), jnp.float32),
        ],
    )
    def k(g_hbm, src_hbm, dst_hbm, out_hbm,
          sidx_v, didx_v, sidx_t, didx_t, rows_v, rows_t, stage_v, sem, acc_sh):
        c = lax.axis_index("c")
        s = lax.axis_index("s")
        wid = s * _NC + c

        def zrow(i, carry):
            r = i // (h // 16)
            col = (i % (h // 16)) * 16
            stage_v[r, pl.ds(col, 16)] = jnp.zeros((16,), jnp.float32)
            return carry

        lax.fori_loop(0, stage_rows * (h // 16), zrow, 0)
        row0 = s * rows_per_tile
        for j in range(n_stage):
            pltpu.sync_copy(stage_v,
                            acc_sh.at[pl.ds(row0 + j * stage_rows, stage_rows)])
        plsc.subcore_barrier()

        base = wid * epw

        def step(i, carry):
            b = base + i * _C
            pltpu.sync_copy(src_hbm.at[pl.ds(b, _C)], sidx_v)
            pltpu.sync_copy(dst_hbm.at[pl.ds(b, _C)], didx_v)
            pltpu.async_copy(g_hbm.at[sidx_v], rows_v, sem).wait()
            pltpu.sync_copy(rows_v, acc_sh.at[didx_v], add=True)
            return carry

        lax.fori_loop(0, full, step, 0)
        if tail:
            b = base + full * _C
            pltpu.sync_copy(src_hbm.at[pl.ds(b, tail)], sidx_t)
            pltpu.sync_copy(dst_hbm.at[pl.ds(b, tail)], didx_t)
            pltpu.async_copy(g_hbm.at[sidx_t], rows_t, sem).wait()
            pltpu.sync_copy(rows_t, acc_sh.at[didx_t], add=True)
        plsc.subcore_barrier()

        for j in range(n_stage):
            r = row0 + j * stage_rows
            pltpu.sync_copy(acc_sh.at[pl.ds(r, stage_rows)], stage_v)
            pltpu.sync_copy(stage_v, out_hbm.at[c, pl.ds(r, stage_rows)])

    return k(g, src, dst)


def _tc_first(pa, pb, x, w1):
    n = x.shape[0]

    def body(pa_r, pb_r, x_r, w_r, dinv_r, g_r):
        deg = pa_r[...] + pb_r[...] + 1.0
        dinv = lax.rsqrt(deg)
        dinv_r[...] = dinv
        g_r[...] = jnp.dot(x_r[...], w_r[...],
                           preferred_element_type=jnp.float32) * dinv

    return pl.pallas_call(
        body,
        out_shape=(
            jax.ShapeDtypeStruct((n, 1), jnp.float32),
            jax.ShapeDtypeStruct((n, w1.shape[1]), jnp.float32),
        ),
    )(pa, pb, x, w1)


def _tc_mid(acc, g, dinv, b, w_next):
    n, hdim = g.shape

    def body(acc_r, g_r, dinv_r, b_r, w_r, out_r):
        dinv = dinv_r[...]
        asum = acc_r[0, pl.ds(0, n), :] + acc_r[1, pl.ds(0, n), :]
        hval = dinv * (asum + g_r[...]) + b_r[...]
        hval = jnp.maximum(hval, 0.0)
        out_r[...] = jnp.dot(hval, w_r[...],
                             preferred_element_type=jnp.float32) * dinv

    return pl.pallas_call(
        body,
        out_shape=jax.ShapeDtypeStruct((n, w_next.shape[1]), jnp.float32),
    )(acc, g, dinv, b, w_next)


def _tc_final(acc, g, dinv, b3, batch2, wl1, bl1, wl2, bl2):
    n, hdim = g.shape
    ngr = _NUM_GRAPHS

    def body(acc_r, g_r, dinv_r, b_r, batch_r, wl1_r, bl1_r, wl2_r, bl2_r,
             out_r):
        asum = acc_r[0, pl.ds(0, n), :] + acc_r[1, pl.ds(0, n), :]
        h3 = dinv_r[...] * (asum + g_r[...]) + b_r[...]
        gids = lax.broadcasted_iota(jnp.int32, (ngr, n), 0)
        onehot_t = (gids == batch_r[...]).astype(jnp.float32)
        pooled = jnp.dot(onehot_t, h3, preferred_element_type=jnp.float32)
        t = jnp.maximum(
            jnp.dot(pooled, wl1_r[...], preferred_element_type=jnp.float32)
            + bl1_r[...], 0.0)
        out_r[...] = (jnp.dot(t, wl2_r[...],
                              preferred_element_type=jnp.float32) + bl2_r[...])

    return pl.pallas_call(
        body,
        out_shape=jax.ShapeDtypeStruct((ngr, 1), jnp.float32),
    )(acc, g, dinv, b3, batch2, wl1, bl1, wl2, bl2)


def kernel(x, edge_index, batch, W1, b1, W2, b2, W3, b3, Wl1, bl1, Wl2, bl2):
    n = x.shape[0]
    src = edge_index[0]
    dst = edge_index[1]

    n_pad = ((n + _NW * 16 - 1) // (_NW * 16)) * (_NW * 16)
    deg_parts = _sc_deg(dst, n_pad)
    pa = deg_parts[0, :n].reshape(n, 1)
    pb = deg_parts[1, :n].reshape(n, 1)

    dinv, g1 = _tc_first(pa, pb, x, W1)
    acc1 = _sc_propagate(g1, src, dst, n_pad)
    g2 = _tc_mid(acc1, g1, dinv, b1.reshape(1, -1), W2)
    acc2 = _sc_propagate(g2, src, dst, n_pad)
    g3 = _tc_mid(acc2, g2, dinv, b2.reshape(1, -1), W3)
    acc3 = _sc_propagate(g3, src, dst, n_pad)
    out = _tc_final(acc3, g3, dinv, b3.reshape(1, -1), batch.reshape(1, n),
                    Wl1, bl1.reshape(1, -1), Wl2, bl2.reshape(1, 1))
    return out


# speedup vs baseline: 25.9669x; 1.5429x over previous
import functools

import jax
import jax.numpy as jnp
from jax import lax
from jax.experimental import pallas as pl
from jax.experimental.pallas import tpu as pltpu
from jax.experimental.pallas import tpu_sc as plsc

_NUM_GRAPHS = 64
_NC = 2
_NS = 16
_NW = _NC * _NS
_C = 80


def _mesh():
    return plsc.VectorSubcoreMesh(core_axis_name="c", subcore_axis_name="s")


def _fill_1d(ref, n, val):
    def body(i, carry):
        ref[pl.ds(i * 16, 16)] = jnp.full((16,), val, jnp.float32)
        return carry
    lax.fori_loop(0, n // 16, body, 0)


def _sc_deg(dst2d, n_pad):
    nrows, c2 = dst2d.shape
    assert nrows % _NW == 0
    rpt = nrows // _NW
    per_tile = n_pad // _NS
    depth = 4

    @functools.partial(
        pl.kernel,
        out_type=jax.ShapeDtypeStruct((_NC, n_pad), jnp.float32),
        mesh=_mesh(),
        compiler_params=pltpu.CompilerParams(use_tc_tiling_on_sc=False),
        scratch_types=[
            pltpu.VMEM((rpt, c2), jnp.int32),
            pltpu.VMEM((c2,), jnp.float32),
            pltpu.VMEM((per_tile,), jnp.float32),
            pltpu.SemaphoreType.DMA,
            pltpu.SemaphoreType.DMA,
            pltpu.VMEM_SHARED((n_pad,), jnp.float32),
        ],
    )
    def k(dst_hbm, out_hbm, didx, ones_v, stage_v, sem_i, sem_s, acc_sh):
        c = lax.axis_index("c")
        s = lax.axis_index("s")
        wid = s * _NC + c
        cp_i = pltpu.async_copy(dst_hbm.at[pl.ds(wid * rpt, rpt)], didx, sem_i)
        _fill_1d(ones_v, c2, 1.0)
        _fill_1d(stage_v, per_tile, 0.0)
        pltpu.sync_copy(stage_v, acc_sh.at[pl.ds(s * per_tile, per_tile)])
        plsc.subcore_barrier()
        cp_i.wait()

        def sca(j):
            pltpu.async_copy(ones_v, acc_sh.at[didx.at[j]], sem_s, add=True)

        def sca_wait():
            pltpu.make_async_copy(ones_v, acc_sh.at[didx.at[0]], sem_s).wait()

        for j in range(depth):
            sca(j)

        def step(j, carry):
            sca_wait()
            sca(j + depth)
            return carry

        lax.fori_loop(0, rpt - depth, step, 0)
        for _ in range(depth):
            sca_wait()
        plsc.subcore_barrier()
        pltpu.sync_copy(acc_sh.at[pl.ds(s * per_tile, per_tile)], stage_v)
        pltpu.sync_copy(stage_v, out_hbm.at[c, pl.ds(s * per_tile, per_tile)])

    return k(dst2d)


def _sc_propagate(g, src2d, dst2d, n_pad):
    n, h = g.shape
    nrows, c2 = src2d.shape
    assert nrows % _NW == 0
    steps = nrows // _NW
    assert steps >= 3 and steps % 2 == 1
    pairs = (steps - 3) // 2
    rows_per_tile = n_pad // _NS
    stage_rows = 128
    n_stage = rows_per_tile // stage_rows

    @functools.partial(
        pl.kernel,
        out_type=jax.ShapeDtypeStruct((_NC, n_pad, h), jnp.float32),
        mesh=_mesh(),
        compiler_params=pltpu.CompilerParams(use_tc_tiling_on_sc=False),
        scratch_types=[
            pltpu.VMEM((steps, c2), jnp.int32),
            pltpu.VMEM((steps, c2), jnp.int32),
            pltpu.VMEM((c2, h), jnp.float32),
            pltpu.VMEM((c2, h), jnp.float32),
            pltpu.VMEM((stage_rows, h), jnp.float32),
            pltpu.SemaphoreType.DMA,
            pltpu.SemaphoreType.DMA,
            pltpu.SemaphoreType.DMA,
            pltpu.VMEM_SHARED((n_pad, h), jnp.float32),
        ],
    )
    def k(g_hbm, src_hbm, dst_hbm, out_hbm,
          sidx, didx, ra, rb, stage_v, sem_i, sem_g, sem_s, acc_sh):
        c = lax.axis_index("c")
        s = lax.axis_index("s")
        wid = s * _NC + c
        r0 = wid * steps
        cp_si = pltpu.async_copy(src_hbm.at[pl.ds(r0, steps)], sidx, sem_i)
        cp_di = pltpu.async_copy(dst_hbm.at[pl.ds(r0, steps)], didx, sem_i)

        def zrow(i, carry):
            r = i // (h // 16)
            col = (i % (h // 16)) * 16
            stage_v[r, pl.ds(col, 16)] = jnp.zeros((16,), jnp.float32)
            return carry

        lax.fori_loop(0, stage_rows * (h // 16), zrow, 0)
        row0 = s * rows_per_tile
        for j in range(n_stage):
            pltpu.sync_copy(stage_v,
                            acc_sh.at[pl.ds(row0 + j * stage_rows, stage_rows)])
        plsc.subcore_barrier()
        cp_si.wait()
        cp_di.wait()

        def gat(j, buf):
            pltpu.async_copy(g_hbm.at[sidx.at[j]], buf, sem_g)

        def gat_wait(buf):
            pltpu.make_async_copy(g_hbm.at[sidx.at[0]], buf, sem_g).wait()

        def sca(j, buf):
            pltpu.async_copy(buf, acc_sh.at[didx.at[j]], sem_s, add=True)

        def sca_wait(buf):
            pltpu.make_async_copy(buf, acc_sh.at[didx.at[0]], sem_s).wait()

        gat(0, ra)
        gat_wait(ra)
        gat(1, rb)
        sca(0, ra)

        def pair(i, carry):
            j1 = 2 * i + 1
            gat_wait(rb)
            sca_wait(ra)
            gat(j1 + 1, ra)
            sca(j1, rb)
            gat_wait(ra)
            sca_wait(rb)
            gat(j1 + 2, rb)
            sca(j1 + 1, ra)
            return carry

        lax.fori_loop(0, pairs, pair, 0)
        gat_wait(rb)
        sca_wait(ra)
        gat(steps - 1, ra)
        sca(steps - 2, rb)
        gat_wait(ra)
        sca_wait(rb)
        sca(steps - 1, ra)
        sca_wait(ra)
        plsc.subcore_barrier()

        for j in range(n_stage):
            r = row0 + j * stage_rows
            pltpu.sync_copy(acc_sh.at[pl.ds(r, stage_rows)], stage_v)
            pltpu.sync_copy(stage_v, out_hbm.at[c, pl.ds(r, stage_rows)])

    return k(g, src2d, dst2d)


def _tc_first(pa, pb, x, w1):
    n = x.shape[0]

    def body(pa_r, pb_r, x_r, w_r, dinv_r, g_r):
        deg = pa_r[...] + pb_r[...] + 1.0
        dinv = lax.rsqrt(deg)
        dinv_r[...] = dinv
        g_r[...] = jnp.dot(x_r[...], w_r[...],
                           preferred_element_type=jnp.float32) * dinv

    return pl.pallas_call(
        body,
        out_shape=(
            jax.ShapeDtypeStruct((n, 1), jnp.float32),
            jax.ShapeDtypeStruct((n, w1.shape[1]), jnp.float32),
        ),
    )(pa, pb, x, w1)


def _tc_mid(acc, g, dinv, b, w_next):
    n, hdim = g.shape

    def body(acc_r, g_r, dinv_r, b_r, w_r, out_r):
        dinv = dinv_r[...]
        asum = acc_r[0, pl.ds(0, n), :] + acc_r[1, pl.ds(0, n), :]
        hval = dinv * (asum + g_r[...]) + b_r[...]
        hval = jnp.maximum(hval, 0.0)
        out_r[...] = jnp.dot(hval, w_r[...],
                             preferred_element_type=jnp.float32) * dinv

    return pl.pallas_call(
        body,
        out_shape=jax.ShapeDtypeStruct((n, w_next.shape[1]), jnp.float32),
    )(acc, g, dinv, b, w_next)


def _tc_final(acc, g, dinv, b3, batch2, wl1, bl1, wl2, bl2):
    n, hdim = g.shape
    ngr = _NUM_GRAPHS

    def body(acc_r, g_r, dinv_r, b_r, batch_r, wl1_r, bl1_r, wl2_r, bl2_r,
             out_r):
        asum = acc_r[0, pl.ds(0, n), :] + acc_r[1, pl.ds(0, n), :]
        h3 = dinv_r[...] * (asum + g_r[...]) + b_r[...]
        gids = lax.broadcasted_iota(jnp.int32, (ngr, n), 0)
        onehot_t = (gids == batch_r[...]).astype(jnp.float32)
        pooled = jnp.dot(onehot_t, h3, preferred_element_type=jnp.float32)
        t = jnp.maximum(
            jnp.dot(pooled, wl1_r[...], preferred_element_type=jnp.float32)
            + bl1_r[...], 0.0)
        out_r[...] = (jnp.dot(t, wl2_r[...],
                              preferred_element_type=jnp.float32) + bl2_r[...])

    return pl.pallas_call(
        body,
        out_shape=jax.ShapeDtypeStruct((ngr, 1), jnp.float32),
    )(acc, g, dinv, b3, batch2, wl1, bl1, wl2, bl2)


def kernel(x, edge_index, batch, W1, b1, W2, b2, W3, b3, Wl1, bl1, Wl2, bl2):
    n = x.shape[0]
    e = edge_index.shape[1]
    src2d = edge_index[0].reshape(e // _C, _C)
    dst2d = edge_index[1].reshape(e // _C, _C)

    n_pad = ((n + _NW * 16 - 1) // (_NW * 16)) * (_NW * 16)
    deg_parts = _sc_deg(dst2d, n_pad)
    pa = deg_parts[0, :n].reshape(n, 1)
    pb = deg_parts[1, :n].reshape(n, 1)

    dinv, g1 = _tc_first(pa, pb, x, W1)
    acc1 = _sc_propagate(g1, src2d, dst2d, n_pad)
    g2 = _tc_mid(acc1, g1, dinv, b1.reshape(1, -1), W2)
    acc2 = _sc_propagate(g2, src2d, dst2d, n_pad)
    g3 = _tc_mid(acc2, g2, dinv, b2.reshape(1, -1), W3)
    acc3 = _sc_propagate(g3, src2d, dst2d, n_pad)
    out = _tc_final(acc3, g3, dinv, b3.reshape(1, -1), batch.reshape(1, n),
                    Wl1, bl1.reshape(1, -1), Wl2, bl2.reshape(1, 1))
    return out


# speedup vs baseline: 39.5824x; 1.5243x over previous
import functools

import jax
import jax.numpy as jnp
from jax import lax
from jax.experimental import pallas as pl
from jax.experimental.pallas import tpu as pltpu
from jax.experimental.pallas import tpu_sc as plsc

_NUM_GRAPHS = 64
_NC = 2
_NS = 16
_NW = _NC * _NS
_C = 128
_NBUF = 4


def _mesh():
    return plsc.VectorSubcoreMesh(core_axis_name="c", subcore_axis_name="s")


def _fill_1d(ref, n, val):
    def body(i, carry):
        ref[pl.ds(i * 16, 16)] = jnp.full((16,), val, jnp.float32)
        return carry
    lax.fori_loop(0, n // 16, body, 0)


def _sc_deg(dst2d, n_pad):
    nrows, c2 = dst2d.shape
    assert nrows % _NW == 0
    rpt = nrows // _NW
    per_tile = n_pad // _NS
    depth = 4

    @functools.partial(
        pl.kernel,
        out_type=jax.ShapeDtypeStruct((_NC, n_pad), jnp.float32),
        mesh=_mesh(),
        compiler_params=pltpu.CompilerParams(use_tc_tiling_on_sc=False),
        scratch_types=[
            pltpu.VMEM((rpt, c2), jnp.int32),
            pltpu.VMEM((c2,), jnp.float32),
            pltpu.VMEM((per_tile,), jnp.float32),
            pltpu.SemaphoreType.DMA,
            pltpu.SemaphoreType.DMA,
            pltpu.VMEM_SHARED((n_pad,), jnp.float32),
        ],
    )
    def k(dst_hbm, out_hbm, didx, ones_v, stage_v, sem_i, sem_s, acc_sh):
        c = lax.axis_index("c")
        s = lax.axis_index("s")
        wid = s * _NC + c
        cp_i = pltpu.async_copy(dst_hbm.at[pl.ds(wid * rpt, rpt)], didx, sem_i)
        _fill_1d(ones_v, c2, 1.0)
        _fill_1d(stage_v, per_tile, 0.0)
        pltpu.sync_copy(stage_v, acc_sh.at[pl.ds(s * per_tile, per_tile)])
        plsc.subcore_barrier()
        cp_i.wait()

        def sca(j):
            pltpu.async_copy(ones_v, acc_sh.at[didx.at[j]], sem_s, add=True)

        def sca_wait():
            pltpu.make_async_copy(ones_v, acc_sh.at[didx.at[0]], sem_s).wait()

        for j in range(depth):
            sca(j)

        def step(j, carry):
            sca_wait()
            sca(j + depth)
            return carry

        lax.fori_loop(0, rpt - depth, step, 0)
        for _ in range(depth):
            sca_wait()
        plsc.subcore_barrier()
        pltpu.sync_copy(acc_sh.at[pl.ds(s * per_tile, per_tile)], stage_v)
        pltpu.sync_copy(stage_v, out_hbm.at[c, pl.ds(s * per_tile, per_tile)])

    return k(dst2d)


def _sc_propagate(g, src2d, dst2d, n_pad):
    n, h = g.shape
    nrows, c2 = src2d.shape
    assert nrows % _NW == 0
    steps = nrows // _NW
    assert steps % _NBUF == 0 and steps // _NBUF >= 2
    nblocks = steps // _NBUF
    rows_per_tile = n_pad // _NS
    stage_rows = 128
    n_stage = rows_per_tile // stage_rows

    @functools.partial(
        pl.kernel,
        out_type=jax.ShapeDtypeStruct((_NC, n_pad, h), jnp.float32),
        mesh=_mesh(),
        compiler_params=pltpu.CompilerParams(use_tc_tiling_on_sc=False),
        scratch_types=[
            pltpu.VMEM((steps, c2), jnp.int32),
            pltpu.VMEM((steps, c2), jnp.int32),
            [pltpu.VMEM((c2, h), jnp.float32)] * _NBUF,
            pltpu.VMEM((stage_rows, h), jnp.float32),
            pltpu.SemaphoreType.DMA,
            [pltpu.SemaphoreType.DMA] * _NBUF,
            [pltpu.SemaphoreType.DMA] * _NBUF,
            pltpu.VMEM_SHARED((n_pad, h), jnp.float32),
        ],
    )
    def k(g_hbm, src_hbm, dst_hbm, out_hbm,
          sidx, didx, bufs, stage_v, sem_i, sems_g, sems_s, acc_sh):
        c = lax.axis_index("c")
        s = lax.axis_index("s")
        wid = s * _NC + c
        r0 = wid * steps
        cp_si = pltpu.async_copy(src_hbm.at[pl.ds(r0, steps)], sidx, sem_i)
        cp_di = pltpu.async_copy(dst_hbm.at[pl.ds(r0, steps)], didx, sem_i)

        def zrow(i, carry):
            r = i // (h // 16)
            col = (i % (h // 16)) * 16
            stage_v[r, pl.ds(col, 16)] = jnp.zeros((16,), jnp.float32)
            return carry

        lax.fori_loop(0, stage_rows * (h // 16), zrow, 0)
        row0 = s * rows_per_tile
        for j in range(n_stage):
            pltpu.sync_copy(stage_v,
                            acc_sh.at[pl.ds(row0 + j * stage_rows, stage_rows)])
        plsc.subcore_barrier()
        cp_si.wait()
        cp_di.wait()

        def gat(j, k):
            pltpu.async_copy(g_hbm.at[sidx.at[j]], bufs[k], sems_g[k])

        def gat_wait(k):
            pltpu.make_async_copy(g_hbm.at[sidx.at[0]], bufs[k],
                                  sems_g[k]).wait()

        def sca(j, k):
            pltpu.async_copy(bufs[k], acc_sh.at[didx.at[j]], sems_s[k],
                             add=True)

        def sca_wait(k):
            pltpu.make_async_copy(bufs[k], acc_sh.at[didx.at[0]],
                                  sems_s[k]).wait()

        for k in range(_NBUF):
            gat(k, k)

        def block(i, carry):
            j0 = i * _NBUF
            for k in range(_NBUF):
                gat_wait(k)
                sca(j0 + k, k)
            for k in range(_NBUF):
                sca_wait(k)
                gat(j0 + _NBUF + k, k)
            return carry

        lax.fori_loop(0, nblocks - 1, block, 0)
        j0 = (nblocks - 1) * _NBUF
        for k in range(_NBUF):
            gat_wait(k)
            sca(j0 + k, k)
        for k in range(_NBUF):
            sca_wait(k)
        plsc.subcore_barrier()

        for j in range(n_stage):
            r = row0 + j * stage_rows
            pltpu.sync_copy(acc_sh.at[pl.ds(r, stage_rows)], stage_v)
            pltpu.sync_copy(stage_v, out_hbm.at[c, pl.ds(r, stage_rows)])

    return k(g, src2d, dst2d)


def _tc_first(pa, pb, x, w1):
    n = x.shape[0]

    def body(pa_r, pb_r, x_r, w_r, dinv_r, g_r):
        deg = pa_r[...] + pb_r[...] + 1.0
        dinv = lax.rsqrt(deg)
        dinv_r[...] = dinv
        g_r[...] = jnp.dot(x_r[...], w_r[...],
                           preferred_element_type=jnp.float32) * dinv

    return pl.pallas_call(
        body,
        out_shape=(
            jax.ShapeDtypeStruct((n, 1), jnp.float32),
            jax.ShapeDtypeStruct((n, w1.shape[1]), jnp.float32),
        ),
    )(pa, pb, x, w1)


def _tc_mid(acc, g, dinv, b, w_next):
    n, hdim = g.shape

    def body(acc_r, g_r, dinv_r, b_r, w_r, out_r):
        dinv = dinv_r[...]
        asum = acc_r[0, pl.ds(0, n), :] + acc_r[1, pl.ds(0, n), :]
        hval = dinv * (asum + g_r[...]) + b_r[...]
        hval = jnp.maximum(hval, 0.0)
        out_r[...] = jnp.dot(hval, w_r[...],
                             preferred_element_type=jnp.float32) * dinv

    return pl.pallas_call(
        body,
        out_shape=jax.ShapeDtypeStruct((n, w_next.shape[1]), jnp.float32),
    )(acc, g, dinv, b, w_next)


def _tc_final(acc, g, dinv, b3, batch2, wl1, bl1, wl2, bl2):
    n, hdim = g.shape
    ngr = _NUM_GRAPHS

    def body(acc_r, g_r, dinv_r, b_r, batch_r, wl1_r, bl1_r, wl2_r, bl2_r,
             out_r):
        asum = acc_r[0, pl.ds(0, n), :] + acc_r[1, pl.ds(0, n), :]
        h3 = dinv_r[...] * (asum + g_r[...]) + b_r[...]
        gids = lax.broadcasted_iota(jnp.int32, (ngr, n), 0)
        onehot_t = (gids == batch_r[...]).astype(jnp.float32)
        pooled = jnp.dot(onehot_t, h3, preferred_element_type=jnp.float32)
        t = jnp.maximum(
            jnp.dot(pooled, wl1_r[...], preferred_element_type=jnp.float32)
            + bl1_r[...], 0.0)
        out_r[...] = (jnp.dot(t, wl2_r[...],
                              preferred_element_type=jnp.float32) + bl2_r[...])

    return pl.pallas_call(
        body,
        out_shape=jax.ShapeDtypeStruct((ngr, 1), jnp.float32),
    )(acc, g, dinv, b3, batch2, wl1, bl1, wl2, bl2)


def kernel(x, edge_index, batch, W1, b1, W2, b2, W3, b3, Wl1, bl1, Wl2, bl2):
    n = x.shape[0]
    e = edge_index.shape[1]

    n_pad = ((n + _NW * 16 - 1) // (_NW * 16)) * (_NW * 16)
    if n_pad == n:
        n_pad += _NW * 16

    per_tile = -(-e // _NW)
    steps_raw = -(-per_tile // _C)
    steps = -(-steps_raw // _NBUF) * _NBUF
    e_pad = _NW * steps * _C
    pad = e_pad - e
    idxp = jnp.arange(pad, dtype=edge_index.dtype)
    srcp = jnp.concatenate([edge_index[0], idxp % n])
    dstp = jnp.concatenate([edge_index[1], n + idxp % (n_pad - n)])
    src2d = srcp.reshape(e_pad // _C, _C)
    dst2d = dstp.reshape(e_pad // _C, _C)
    deg_parts = _sc_deg(dst2d, n_pad)
    pa = deg_parts[0, :n].reshape(n, 1)
    pb = deg_parts[1, :n].reshape(n, 1)

    dinv, g1 = _tc_first(pa, pb, x, W1)
    acc1 = _sc_propagate(g1, src2d, dst2d, n_pad)
    g2 = _tc_mid(acc1, g1, dinv, b1.reshape(1, -1), W2)
    acc2 = _sc_propagate(g2, src2d, dst2d, n_pad)
    g3 = _tc_mid(acc2, g2, dinv, b2.reshape(1, -1), W3)
    acc3 = _sc_propagate(g3, src2d, dst2d, n_pad)
    out = _tc_final(acc3, g3, dinv, b3.reshape(1, -1), batch.reshape(1, n),
                    Wl1, bl1.reshape(1, -1), Wl2, bl2.reshape(1, 1))
    return out


# speedup vs baseline: 39.7589x; 1.0045x over previous
import functools

import jax
import jax.numpy as jnp
from jax import lax
from jax.experimental import pallas as pl
from jax.experimental.pallas import tpu as pltpu
from jax.experimental.pallas import tpu_sc as plsc

_NUM_GRAPHS = 64
_NC = 2
_NS = 16
_NW = _NC * _NS
_C = 128
_NBUF = 5


def _mesh():
    return plsc.VectorSubcoreMesh(core_axis_name="c", subcore_axis_name="s")


def _fill_1d(ref, n, val):
    def body(i, carry):
        ref[pl.ds(i * 16, 16)] = jnp.full((16,), val, jnp.float32)
        return carry
    lax.fori_loop(0, n // 16, body, 0)


def _sc_deg(dst2d, n_pad):
    nrows, c2 = dst2d.shape
    assert nrows % _NW == 0
    rpt = nrows // _NW
    per_tile = n_pad // _NS
    depth = 4

    @functools.partial(
        pl.kernel,
        out_type=jax.ShapeDtypeStruct((_NC, n_pad), jnp.float32),
        mesh=_mesh(),
        compiler_params=pltpu.CompilerParams(use_tc_tiling_on_sc=False),
        scratch_types=[
            pltpu.VMEM((rpt, c2), jnp.int32),
            pltpu.VMEM((c2,), jnp.float32),
            pltpu.VMEM((per_tile,), jnp.float32),
            pltpu.SemaphoreType.DMA,
            pltpu.SemaphoreType.DMA,
            pltpu.VMEM_SHARED((n_pad,), jnp.float32),
        ],
    )
    def k(dst_hbm, out_hbm, didx, ones_v, stage_v, sem_i, sem_s, acc_sh):
        c = lax.axis_index("c")
        s = lax.axis_index("s")
        wid = s * _NC + c
        cp_i = pltpu.async_copy(dst_hbm.at[pl.ds(wid * rpt, rpt)], didx, sem_i)
        _fill_1d(ones_v, c2, 1.0)
        _fill_1d(stage_v, per_tile, 0.0)
        pltpu.sync_copy(stage_v, acc_sh.at[pl.ds(s * per_tile, per_tile)])
        plsc.subcore_barrier()
        cp_i.wait()

        def sca(j):
            pltpu.async_copy(ones_v, acc_sh.at[didx.at[j]], sem_s, add=True)

        def sca_wait():
            pltpu.make_async_copy(ones_v, acc_sh.at[didx.at[0]], sem_s).wait()

        for j in range(depth):
            sca(j)

        def step(j, carry):
            sca_wait()
            sca(j + depth)
            return carry

        lax.fori_loop(0, rpt - depth, step, 0)
        for _ in range(depth):
            sca_wait()
        plsc.subcore_barrier()
        pltpu.sync_copy(acc_sh.at[pl.ds(s * per_tile, per_tile)], stage_v)
        pltpu.sync_copy(stage_v, out_hbm.at[c, pl.ds(s * per_tile, per_tile)])

    return k(dst2d)


def _sc_propagate(g, src2d, dst2d, n_pad):
    n, h = g.shape
    nrows, c2 = src2d.shape
    assert nrows % _NW == 0
    steps = nrows // _NW
    assert steps % _NBUF == 0 and steps // _NBUF >= 2
    nblocks = steps // _NBUF
    rows_per_tile = n_pad // _NS
    stage_rows = 64
    n_stage = rows_per_tile // stage_rows

    @functools.partial(
        pl.kernel,
        out_type=jax.ShapeDtypeStruct((_NC, n_pad, h), jnp.float32),
        mesh=_mesh(),
        compiler_params=pltpu.CompilerParams(use_tc_tiling_on_sc=False),
        scratch_types=[
            pltpu.VMEM((steps, c2), jnp.int32),
            pltpu.VMEM((steps, c2), jnp.int32),
            [pltpu.VMEM((c2, h), jnp.float32)] * _NBUF,
            pltpu.VMEM((stage_rows, h), jnp.float32),
            pltpu.SemaphoreType.DMA,
            [pltpu.SemaphoreType.DMA] * _NBUF,
            [pltpu.SemaphoreType.DMA] * _NBUF,
            pltpu.VMEM_SHARED((n_pad, h), jnp.float32),
        ],
    )
    def k(g_hbm, src_hbm, dst_hbm, out_hbm,
          sidx, didx, bufs, stage_v, sem_i, sems_g, sems_s, acc_sh):
        c = lax.axis_index("c")
        s = lax.axis_index("s")
        wid = s * _NC + c
        r0 = wid * steps
        cp_si = pltpu.async_copy(src_hbm.at[pl.ds(r0, steps)], sidx, sem_i)
        cp_di = pltpu.async_copy(dst_hbm.at[pl.ds(r0, steps)], didx, sem_i)

        def zrow(i, carry):
            r = i // (h // 16)
            col = (i % (h // 16)) * 16
            stage_v[r, pl.ds(col, 16)] = jnp.zeros((16,), jnp.float32)
            return carry

        lax.fori_loop(0, stage_rows * (h // 16), zrow, 0)
        row0 = s * rows_per_tile
        for j in range(n_stage):
            pltpu.sync_copy(stage_v,
                            acc_sh.at[pl.ds(row0 + j * stage_rows, stage_rows)])
        plsc.subcore_barrier()
        cp_si.wait()
        cp_di.wait()

        def gat(j, k):
            pltpu.async_copy(g_hbm.at[sidx.at[j]], bufs[k], sems_g[k])

        def gat_wait(k):
            pltpu.make_async_copy(g_hbm.at[sidx.at[0]], bufs[k],
                                  sems_g[k]).wait()

        def sca(j, k):
            pltpu.async_copy(bufs[k], acc_sh.at[didx.at[j]], sems_s[k],
                             add=True)

        def sca_wait(k):
            pltpu.make_async_copy(bufs[k], acc_sh.at[didx.at[0]],
                                  sems_s[k]).wait()

        for k in range(_NBUF):
            gat(k, k)

        def block(i, carry):
            j0 = i * _NBUF
            for k in range(_NBUF):
                gat_wait(k)
                sca(j0 + k, k)
            for k in range(_NBUF):
                sca_wait(k)
                gat(j0 + _NBUF + k, k)
            return carry

        lax.fori_loop(0, nblocks - 1, block, 0)
        j0 = (nblocks - 1) * _NBUF
        for k in range(_NBUF):
            gat_wait(k)
            sca(j0 + k, k)
        for k in range(_NBUF):
            sca_wait(k)
        plsc.subcore_barrier()

        for j in range(n_stage):
            r = row0 + j * stage_rows
            pltpu.sync_copy(acc_sh.at[pl.ds(r, stage_rows)], stage_v)
            pltpu.sync_copy(stage_v, out_hbm.at[c, pl.ds(r, stage_rows)])

    return k(g, src2d, dst2d)


def _tc_xw(x, w1):
    n = x.shape[0]

    def body(x_r, w_r, xw_r):
        xw_r[...] = jnp.dot(x_r[...], w_r[...],
                            preferred_element_type=jnp.float32)

    return pl.pallas_call(
        body,
        out_shape=jax.ShapeDtypeStruct((n, w1.shape[1]), jnp.float32),
    )(x, w1)


def _tc_first(pa, pb, xw):
    n = xw.shape[0]

    def body(pa_r, pb_r, xw_r, dinv_r, g_r):
        deg = pa_r[...] + pb_r[...] + 1.0
        dinv = lax.rsqrt(deg)
        dinv_r[...] = dinv
        g_r[...] = xw_r[...] * dinv

    return pl.pallas_call(
        body,
        out_shape=(
            jax.ShapeDtypeStruct((n, 1), jnp.float32),
            jax.ShapeDtypeStruct((n, xw.shape[1]), jnp.float32),
        ),
    )(pa, pb, xw)


def _tc_mid(acc, g, dinv, b, w_next):
    n, hdim = g.shape

    def body(acc_r, g_r, dinv_r, b_r, w_r, out_r):
        dinv = dinv_r[...]
        asum = acc_r[0, pl.ds(0, n), :] + acc_r[1, pl.ds(0, n), :]
        hval = dinv * (asum + g_r[...]) + b_r[...]
        hval = jnp.maximum(hval, 0.0)
        out_r[...] = jnp.dot(hval, w_r[...],
                             preferred_element_type=jnp.float32) * dinv

    return pl.pallas_call(
        body,
        out_shape=jax.ShapeDtypeStruct((n, w_next.shape[1]), jnp.float32),
    )(acc, g, dinv, b, w_next)


def _tc_final(acc, g, dinv, b3, batch2, wl1, bl1, wl2, bl2):
    n, hdim = g.shape
    ngr = _NUM_GRAPHS

    def body(acc_r, g_r, dinv_r, b_r, batch_r, wl1_r, bl1_r, wl2_r, bl2_r,
             out_r):
        asum = acc_r[0, pl.ds(0, n), :] + acc_r[1, pl.ds(0, n), :]
        h3 = dinv_r[...] * (asum + g_r[...]) + b_r[...]
        gids = lax.broadcasted_iota(jnp.int32, (ngr, n), 0)
        onehot_t = (gids == batch_r[...]).astype(jnp.float32)
        pooled = jnp.dot(onehot_t, h3, preferred_element_type=jnp.float32)
        t = jnp.maximum(
            jnp.dot(pooled, wl1_r[...], preferred_element_type=jnp.float32)
            + bl1_r[...], 0.0)
        out_r[...] = (jnp.dot(t, wl2_r[...],
                              preferred_element_type=jnp.float32) + bl2_r[...])

    return pl.pallas_call(
        body,
        out_shape=jax.ShapeDtypeStruct((ngr, 1), jnp.float32),
    )(acc, g, dinv, b3, batch2, wl1, bl1, wl2, bl2)


def kernel(x, edge_index, batch, W1, b1, W2, b2, W3, b3, Wl1, bl1, Wl2, bl2):
    n = x.shape[0]
    e = edge_index.shape[1]

    n_pad = ((n + _NW * 16 - 1) // (_NW * 16)) * (_NW * 16)
    if n_pad == n:
        n_pad += _NW * 16

    per_tile = -(-e // _NW)
    steps_raw = -(-per_tile // _C)
    steps = -(-steps_raw // _NBUF) * _NBUF
    e_pad = _NW * steps * _C
    pad = e_pad - e
    idxp = jnp.arange(pad, dtype=edge_index.dtype)
    srcp = jnp.concatenate([edge_index[0], idxp % n])
    dstp = jnp.concatenate([edge_index[1], n + idxp % (n_pad - n)])
    src2d = srcp.reshape(e_pad // _C, _C)
    dst2d = dstp.reshape(e_pad // _C, _C)
    deg_parts = _sc_deg(dst2d, n_pad)
    xw = _tc_xw(x, W1)
    pa = deg_parts[0, :n].reshape(n, 1)
    pb = deg_parts[1, :n].reshape(n, 1)

    dinv, g1 = _tc_first(pa, pb, xw)
    acc1 = _sc_propagate(g1, src2d, dst2d, n_pad)
    g2 = _tc_mid(acc1, g1, dinv, b1.reshape(1, -1), W2)
    acc2 = _sc_propagate(g2, src2d, dst2d, n_pad)
    g3 = _tc_mid(acc2, g2, dinv, b2.reshape(1, -1), W3)
    acc3 = _sc_propagate(g3, src2d, dst2d, n_pad)
    out = _tc_final(acc3, g3, dinv, b3.reshape(1, -1), batch.reshape(1, n),
                    Wl1, bl1.reshape(1, -1), Wl2, bl2.reshape(1, 1))
    return out
